# baseline (speedup 1.0000x reference)
import jax
import jax.numpy as jnp
from jax.experimental import pallas as pl
from jax.experimental.pallas import tpu as pltpu

KS = 3
EPS = 1e-8


def _cconv_kernel(po_ref, pi_ref, ext_ref, f_ref, wf_ref, b_ref, o_ref,
                  wv_ref, scr_ref, fbf_ref, whi_ref, wlo_ref, piT_ref,
                  sc_ref):
    bo = o_ref.shape[0]
    cin = f_ref.shape[1]

    @pl.when(pl.program_id(0) == 0)
    def _prep():
        scale = 2.0 / ext_ref[0, 0]
        sc_ref[0, 0] = scale
        piT_ref[0:3, :] = jnp.transpose(pi_ref[...], (1, 0)) * scale
        fbf_ref[...] = f_ref[...].astype(jnp.bfloat16)
        wf = wf_ref[...]
        whi = wf.astype(jnp.bfloat16)
        whi_ref[...] = whi
        wlo_ref[...] = (wf - whi.astype(jnp.float32)).astype(jnp.bfloat16)

    scale = sc_ref[0, 0]
    pox = po_ref[:, 0:1] * scale
    poy = po_ref[:, 1:2] * scale
    poz = po_ref[:, 2:3] * scale
    relx = piT_ref[0:1, :] - pox
    rely = piT_ref[1:2, :] - poy
    relz = piT_ref[2:3, :] - poz
    sx2 = relx * relx
    sy2 = rely * rely
    sz2 = relz * relz
    r2 = sx2 + sy2 + sz2
    inside = (r2 <= 1.0).astype(jnp.bfloat16)
    linf2 = jnp.maximum(jnp.maximum(sx2, sy2), jnp.maximum(sz2, EPS * EPS))
    s = jnp.sqrt(jnp.maximum(r2, EPS * EPS) / linf2)
    gx = jnp.clip(relx * s + 1.0, 0.0, 2.0).astype(jnp.bfloat16)
    gy = jnp.clip(rely * s + 1.0, 0.0, 2.0).astype(jnp.bfloat16)
    gz = jnp.clip(relz * s + 1.0, 0.0, 2.0).astype(jnp.bfloat16)

    num = jnp.sum(inside.astype(jnp.float32), axis=1, keepdims=True)
    denom = jnp.maximum(num, 1.0)

    wx = (jnp.maximum(1.0 - gx, 0.0), 1.0 - jnp.abs(gx - 1.0),
          jnp.maximum(gx - 1.0, 0.0))
    wy = (jnp.maximum(1.0 - gy, 0.0), 1.0 - jnp.abs(gy - 1.0),
          jnp.maximum(gy - 1.0, 0.0))
    wz = (jnp.maximum(1.0 - gz, 0.0) * inside,
          (1.0 - jnp.abs(gz - 1.0)) * inside,
          jnp.maximum(gz - 1.0, 0.0) * inside)

    for vx in range(KS):
        for vy in range(KS):
            wxy = wx[vx] * wy[vy]
            for vz in range(KS):
                k = (vx * KS + vy) * KS + vz
                wv_ref[k * bo:(k + 1) * bo, :] = wxy * wz[vz]

    tmpstack = jnp.dot(wv_ref[...], fbf_ref[...],
                       preferred_element_type=jnp.float32)
    for k in range(KS * KS * KS):
        scr_ref[:, k * cin:(k + 1) * cin] = (
            tmpstack[k * bo:(k + 1) * bo, :].astype(jnp.bfloat16))

    scr = scr_ref[...]
    acc = (jnp.dot(scr, whi_ref[...], preferred_element_type=jnp.float32) +
           jnp.dot(scr, wlo_ref[...], preferred_element_type=jnp.float32))
    o_ref[...] = jnp.maximum(acc / denom + b_ref[...], 0.0)


def kernel(features, pos_input, pos_output, extents, W, b):
    n_in, cin = features.shape
    n_out = pos_output.shape[0]
    cout = W.shape[-1]
    wf = W.reshape(KS * KS * KS * cin, cout)
    b2 = b.reshape(1, cout)
    ext = extents.reshape(1, 1)

    bo = 128
    grid = (n_out // bo,)
    kcin = KS * KS * KS * cin
    out = pl.pallas_call(
        _cconv_kernel,
        grid=grid,
        in_specs=[
            pl.BlockSpec((bo, 3), lambda o: (o, 0)),
            pl.BlockSpec((n_in, 3), lambda o: (0, 0)),
            pl.BlockSpec((1, 1), lambda o: (0, 0)),
            pl.BlockSpec((n_in, cin), lambda o: (0, 0)),
            pl.BlockSpec((kcin, cout), lambda o: (0, 0)),
            pl.BlockSpec((1, cout), lambda o: (0, 0)),
        ],
        out_specs=pl.BlockSpec((bo, cout), lambda o: (o, 0)),
        out_shape=jax.ShapeDtypeStruct((n_out, cout), jnp.float32),
        scratch_shapes=[pltpu.VMEM((KS * KS * KS * bo, n_in), jnp.bfloat16),
                        pltpu.VMEM((bo, kcin), jnp.bfloat16),
                        pltpu.VMEM((n_in, cin), jnp.bfloat16),
                        pltpu.VMEM((kcin, cout), jnp.bfloat16),
                        pltpu.VMEM((kcin, cout), jnp.bfloat16),
                        pltpu.VMEM((8, n_in), jnp.float32),
                        pltpu.SMEM((1, 1), jnp.float32)],
    )(pos_output, pos_input, ext, features, wf, b2)
    return out


# speedup vs baseline: 1.0252x; 1.0252x over previous
import jax
import jax.numpy as jnp
from jax.experimental import pallas as pl
from jax.experimental.pallas import tpu as pltpu

KS = 3
EPS = 1e-8


def _cconv_kernel(po_ref, piT_ref, f_ref, wf_ref, b_ref, o_ref,
                  wv_ref, scr_ref, fbf_ref, whi_ref, wlo_ref):
    bo = o_ref.shape[0]
    cin = f_ref.shape[1]

    @pl.when(pl.program_id(0) == 0)
    def _prep():
        fbf_ref[...] = f_ref[...].astype(jnp.bfloat16)
        wf = wf_ref[...]
        whi = wf.astype(jnp.bfloat16)
        whi_ref[...] = whi
        wlo_ref[...] = (wf - whi.astype(jnp.float32)).astype(jnp.bfloat16)

    pox = po_ref[:, 0:1]
    poy = po_ref[:, 1:2]
    poz = po_ref[:, 2:3]
    relx = piT_ref[0:1, :] - pox
    rely = piT_ref[1:2, :] - poy
    relz = piT_ref[2:3, :] - poz
    sx2 = relx * relx
    sy2 = rely * rely
    sz2 = relz * relz
    r2 = sx2 + sy2 + sz2
    inside = (r2 <= 1.0).astype(jnp.bfloat16)
    linf2 = jnp.maximum(jnp.maximum(sx2, sy2), jnp.maximum(sz2, EPS * EPS))
    s = jnp.sqrt(jnp.maximum(r2, EPS * EPS) / linf2)
    gx = jnp.clip(relx * s + 1.0, 0.0, 2.0).astype(jnp.bfloat16)
    gy = jnp.clip(rely * s + 1.0, 0.0, 2.0).astype(jnp.bfloat16)
    gz = jnp.clip(relz * s + 1.0, 0.0, 2.0).astype(jnp.bfloat16)

    num = jnp.sum(inside.astype(jnp.float32), axis=1, keepdims=True)
    denom = jnp.maximum(num, 1.0)

    wx = (jnp.maximum(1.0 - gx, 0.0), 1.0 - jnp.abs(gx - 1.0),
          jnp.maximum(gx - 1.0, 0.0))
    wy = (jnp.maximum(1.0 - gy, 0.0), 1.0 - jnp.abs(gy - 1.0),
          jnp.maximum(gy - 1.0, 0.0))
    wz = (jnp.maximum(1.0 - gz, 0.0) * inside,
          (1.0 - jnp.abs(gz - 1.0)) * inside,
          jnp.maximum(gz - 1.0, 0.0) * inside)

    for vx in range(KS):
        for vy in range(KS):
            wxy = wx[vx] * wy[vy]
            for vz in range(KS):
                k = (vx * KS + vy) * KS + vz
                wv_ref[k * bo:(k + 1) * bo, :] = wxy * wz[vz]

    tmpstack = jnp.dot(wv_ref[...], fbf_ref[...],
                       preferred_element_type=jnp.float32)
    for k in range(KS * KS * KS):
        scr_ref[:, k * cin:(k + 1) * cin] = (
            tmpstack[k * bo:(k + 1) * bo, :].astype(jnp.bfloat16))

    scr = scr_ref[...]
    acc = (jnp.dot(scr, whi_ref[...], preferred_element_type=jnp.float32) +
           jnp.dot(scr, wlo_ref[...], preferred_element_type=jnp.float32))
    o_ref[...] = jnp.maximum(acc / denom + b_ref[...], 0.0)


def kernel(features, pos_input, pos_output, extents, W, b):
    n_in, cin = features.shape
    n_out = pos_output.shape[0]
    cout = W.shape[-1]
    scale = 2.0 / extents.reshape(-1)[0]
    po = (pos_output * scale).astype(jnp.float32)
    piT = (pos_input.T * scale).astype(jnp.float32)
    wf = W.reshape(KS * KS * KS * cin, cout)
    b2 = b.reshape(1, cout)

    bo = 128
    grid = (n_out // bo,)
    kcin = KS * KS * KS * cin
    out = pl.pallas_call(
        _cconv_kernel,
        grid=grid,
        in_specs=[
            pl.BlockSpec((bo, 3), lambda o: (o, 0)),
            pl.BlockSpec((3, n_in), lambda o: (0, 0)),
            pl.BlockSpec((n_in, cin), lambda o: (0, 0)),
            pl.BlockSpec((kcin, cout), lambda o: (0, 0)),
            pl.BlockSpec((1, cout), lambda o: (0, 0)),
        ],
        out_specs=pl.BlockSpec((bo, cout), lambda o: (o, 0)),
        out_shape=jax.ShapeDtypeStruct((n_out, cout), jnp.float32),
        scratch_shapes=[pltpu.VMEM((KS * KS * KS * bo, n_in), jnp.bfloat16),
                        pltpu.VMEM((bo, kcin), jnp.bfloat16),
                        pltpu.VMEM((n_in, cin), jnp.bfloat16),
                        pltpu.VMEM((kcin, cout), jnp.bfloat16),
                        pltpu.VMEM((kcin, cout), jnp.bfloat16)],
    )(po, piT, features, wf, b2)
    return out
